# baseline (speedup 1.0000x reference)
import jax
import jax.numpy as jnp
from jax import lax
from jax.experimental import pallas as pl
from jax.experimental.pallas import tpu as pltpu
from jax.experimental.pallas import tpu_sc as plsc

VOCAB = 1000
CPAD = 1024
NBLK = CPAD // 128
N_TOK = 1024 * 50
N_GRP = N_TOK // 8
NC, NS, LANES = 2, 16, 16
NW = NC * NS
R_PER_W = N_TOK // NW
CH = 32
NCH = R_PER_W // CH
NBUF = 3
LCH = 64
NLCH = R_PER_W // LCH


def _lse_body(t_ref, lse_ref):
    t = t_ref[...]
    m = jnp.max(t, axis=1)
    lse_ref[...] = m + jnp.log(jnp.sum(jnp.exp(t - m[:, None]), axis=1))


def _sc_body(tblk, tflat, idx_h, tgt_h, lse_h, out_h, part_h,
             idx_v, tgt_v, fidx_v, lsev, elemv, rows_v, acc_v,
             gsem, wsem, lsem, esem):
    wid = lax.axis_index("s") * NC + lax.axis_index("c")
    base = wid * R_PER_W

    pltpu.sync_copy(idx_h.at[pl.ds(base, R_PER_W)], idx_v)
    pltpu.sync_copy(tgt_h.at[pl.ds(base, R_PER_W)], tgt_v)

    def fidx_body(i, carry):
        s = pl.ds(i * LANES, LANES)
        fidx_v[s] = tgt_v[s] * VOCAB + idx_v[s]
        return carry
    lax.fori_loop(0, R_PER_W // LANES, fidx_body, 0)

    def lse_copy(c):
        s = pl.ds(c * LCH, LCH)
        return pltpu.make_async_copy(lse_h.at[idx_v.at[s]], lsev.at[s], lsem)

    def elem_copy(c):
        s = pl.ds(c * LCH, LCH)
        return pltpu.make_async_copy(tflat.at[fidx_v.at[s]], elemv.at[s], esem)

    def gather_copy(c, b, blk):
        return pltpu.make_async_copy(
            tblk.at[blk].at[idx_v.at[pl.ds(c * CH, CH)]],
            rows_v.at[b, blk], gsem.at[b])

    def write_copies(c, b, tr):
        grp = (base + c * CH) // 8 + tr
        return [pltpu.make_async_copy(
            rows_v.at[b, :, pl.ds(tr * 8, 8), :], out_h.at[grp],
            wsem.at[b])]

    def start_chunk(c, b, drain):
        if drain:
            for tr in range(CH // 8):
                for cp in write_copies(c, b, tr):
                    cp.wait()
        for blk in range(NBLK):
            gather_copy(c, b, blk).start()

    def finish_chunk(c, b):
        for blk in range(NBLK):
            gather_copy(c, b, blk).wait()
        for tr in range(CH // 8):
            for cp in write_copies(c, b, tr):
                cp.start()

    for b in range(NBUF):
        start_chunk(b, b, False)

    def fire_body(c, carry):
        lse_copy(c).start()
        elem_copy(c).start()
        return carry
    lax.fori_loop(0, NLCH, fire_body, 0)

    NMAIN = (NCH - NBUF) // NBUF
    def pipe_body(i, carry):
        for b in range(NBUF):
            c = i * NBUF + b
            finish_chunk(c, b)
            start_chunk(c + NBUF, b, True)
        return carry
    lax.fori_loop(0, NMAIN, pipe_body, 0)

    for c in range(NMAIN * NBUF, NCH):
        b = c % NBUF
        finish_chunk(c, b)
        if c + NBUF < NCH:
            start_chunk(c + NBUF, b, True)
    for b in range(NBUF):
        for tr in range(CH // 8):
            for cp in write_copies(NCH - NBUF + b, b, tr):
                cp.wait()

    def drain_body(c, carry):
        lse_copy(c).wait()
        elem_copy(c).wait()
        return carry
    lax.fori_loop(0, NLCH, drain_body, 0)

    def acc_body(c, acc):
        for g in range(LCH // LANES):
            s = pl.ds(c * LCH + g * LANES, LANES)
            acc = acc + (lsev[s] - elemv[s])
        return acc
    acc = lax.fori_loop(0, NLCH, acc_body, jnp.zeros((LANES,), jnp.float32))

    acc_v[...] = acc
    pltpu.sync_copy(acc_v, part_h.at[wid])


@jax.jit
def kernel(table, idx, targets):
    lse = pl.pallas_call(
        _lse_body,
        out_shape=jax.ShapeDtypeStruct((VOCAB,), jnp.float32),
    )(table)

    idx_f = idx.reshape(-1).astype(jnp.int32)
    tgt_f = targets.reshape(-1).astype(jnp.int32)
    tblk = jnp.pad(table, ((0, 0), (0, CPAD - VOCAB))) \
        .reshape(VOCAB, NBLK, 128).transpose(1, 0, 2)
    tflat = table.T.reshape(-1)

    sc = pl.kernel(
        _sc_body,
        out_type=(jax.ShapeDtypeStruct((N_GRP, NBLK, 8, 128), jnp.float32),
                  jax.ShapeDtypeStruct((NW, LANES), jnp.float32)),
        mesh=plsc.VectorSubcoreMesh(core_axis_name="c", subcore_axis_name="s",
                                    num_cores=NC, num_subcores=NS),
        scratch_types=[
            pltpu.VMEM((R_PER_W,), jnp.int32),
            pltpu.VMEM((R_PER_W,), jnp.int32),
            pltpu.VMEM((R_PER_W,), jnp.int32),
            pltpu.VMEM((R_PER_W,), jnp.float32),
            pltpu.VMEM((R_PER_W,), jnp.float32),
            pltpu.VMEM((NBUF, NBLK, CH, 128), jnp.float32),
            pltpu.VMEM((LANES,), jnp.float32),
            pltpu.SemaphoreType.DMA((NBUF,)),
            pltpu.SemaphoreType.DMA((NBUF,)),
            pltpu.SemaphoreType.DMA,
            pltpu.SemaphoreType.DMA,
        ],
    )
    tiles, part = sc(tblk, tflat, idx_f, tgt_f, lse)
    logits = tiles.transpose(0, 2, 1, 3).reshape(N_TOK, CPAD)[:, :VOCAB]
    loss = jnp.sum(part) / jnp.float32(N_TOK)
    return (logits, loss)
